# Initial kernel scaffold; baseline (speedup 1.0000x reference)
import functools

import jax
import jax.numpy as jnp
from jax.experimental import pallas as pl
from jax.experimental.pallas import tpu as pltpu

NUM_EXPERTS = 8
CHUNK = 256


def _moe_dense_kernel(x_ref, rw_ref, w1_ref, b1_ref, w2_ref, b2_ref,
                      y_ref, aux_ref, comb_ref):
    e = pl.program_id(0)
    n_tok = x_ref.shape[0]

    @pl.when(e == 0)
    def _router():
        xf = x_ref[...]
        logits = jax.lax.dot_general(
            xf, rw_ref[...], (((1,), (1,)), ((), ())),
            precision=jax.lax.Precision.HIGHEST)
        m = jnp.max(logits, axis=-1, keepdims=True)
        p = jnp.exp(logits - m)
        probs = p / jnp.sum(p, axis=-1, keepdims=True)
        iota = jax.lax.broadcasted_iota(jnp.int32, probs.shape, 1)
        m1 = jnp.max(probs, axis=-1, keepdims=True)
        i1 = jnp.min(jnp.where(probs == m1, iota, NUM_EXPERTS),
                     axis=-1, keepdims=True)
        mask1 = iota == i1
        pm = jnp.where(mask1, -jnp.inf, probs)
        m2 = jnp.max(pm, axis=-1, keepdims=True)
        i2 = jnp.min(jnp.where(pm == m2, iota, NUM_EXPERTS),
                     axis=-1, keepdims=True)
        mask2 = iota == i2
        denom = m1 + m2 + 1e-9
        comb_ref[...] = (jnp.where(mask1, m1 / denom, 0.0)
                         + jnp.where(mask2, m2 / denom, 0.0))
        importance = jnp.sum(probs, axis=0) / n_tok
        load = jnp.sum(mask1.astype(jnp.float32), axis=0) / n_tok
        aux_ref[...] = jnp.reshape(
            jnp.sum(importance * load) * NUM_EXPERTS, (1, 1))
        y_ref[...] = jnp.zeros_like(y_ref)

    w1 = w1_ref[0]
    w2 = w2_ref[0]
    b1 = b1_ref[...]
    b2 = b2_ref[...]
    lane = jax.lax.broadcasted_iota(jnp.int32, (n_tok, NUM_EXPERTS), 1)
    w_col = jnp.sum(jnp.where(lane == e, comb_ref[...], 0.0),
                    axis=-1, keepdims=True)

    def body(i, _):
        sl = pl.ds(i * CHUNK, CHUNK)
        xs = x_ref[sl, :]
        h = jax.lax.dot_general(xs, w1, (((1,), (1,)), ((), ()))) + b1
        h = h * jax.nn.sigmoid(h)
        ys = jax.lax.dot_general(h, w2, (((1,), (1,)), ((), ()))) + b2
        y_ref[sl, :] += ys * w_col[sl, :]
        return 0

    jax.lax.fori_loop(0, n_tok // CHUNK, body, 0)


def kernel(x, router_W, fc1_W, fc1_b, fc2_W, fc2_b):
    B, L, D = x.shape
    N = B * L
    E, H = fc1_W.shape[0], fc1_W.shape[1]
    x_flat = x.reshape(N, D)

    y, aux = pl.pallas_call(
        _moe_dense_kernel,
        grid=(E,),
        in_specs=[
            pl.BlockSpec((N, D), lambda e: (0, 0)),
            pl.BlockSpec((E, D), lambda e: (0, 0)),
            pl.BlockSpec((1, H, D), lambda e: (e, 0, 0)),
            pl.BlockSpec((1, H), lambda e: (e, 0)),
            pl.BlockSpec((1, D, H), lambda e: (e, 0, 0)),
            pl.BlockSpec((1, D), lambda e: (e, 0)),
        ],
        out_specs=[
            pl.BlockSpec((N, D), lambda e: (0, 0)),
            pl.BlockSpec((1, 1), lambda e: (0, 0)),
        ],
        out_shape=[
            jax.ShapeDtypeStruct((N, D), jnp.float32),
            jax.ShapeDtypeStruct((1, 1), jnp.float32),
        ],
        scratch_shapes=[pltpu.VMEM((N, NUM_EXPERTS), jnp.float32)],
    )(x_flat, router_W, fc1_W, fc1_b, fc2_W, fc2_b)

    return y.reshape(B, L, D), aux[0, 0]


# speedup vs baseline: 1.3720x; 1.3720x over previous
import functools

import jax
import jax.numpy as jnp
from jax.experimental import pallas as pl
from jax.experimental.pallas import tpu as pltpu

NUM_EXPERTS = 8
CHUNK = 256


def _moe_dense_kernel(x_ref, rw_ref, w1_ref, b1_ref, w2_ref, b2_ref,
                      y_ref, aux_ref, comb_ref):
    e = pl.program_id(0)
    n_tok = x_ref.shape[0]

    @pl.when(e == 0)
    def _router():
        xf = x_ref[...]
        logits = jax.lax.dot_general(
            xf, rw_ref[...], (((1,), (1,)), ((), ())))
        m = jnp.max(logits, axis=-1, keepdims=True)
        p = jnp.exp(logits - m)
        probs = p / jnp.sum(p, axis=-1, keepdims=True)
        iota = jax.lax.broadcasted_iota(jnp.int32, probs.shape, 1)
        m1 = jnp.max(probs, axis=-1, keepdims=True)
        i1 = jnp.min(jnp.where(probs == m1, iota, NUM_EXPERTS),
                     axis=-1, keepdims=True)
        mask1 = iota == i1
        pm = jnp.where(mask1, -jnp.inf, probs)
        m2 = jnp.max(pm, axis=-1, keepdims=True)
        i2 = jnp.min(jnp.where(pm == m2, iota, NUM_EXPERTS),
                     axis=-1, keepdims=True)
        mask2 = iota == i2
        denom = m1 + m2 + 1e-9
        comb_ref[...] = (jnp.where(mask1, m1 / denom, 0.0)
                         + jnp.where(mask2, m2 / denom, 0.0))
        importance = jnp.sum(probs, axis=0) / n_tok
        load = jnp.sum(mask1.astype(jnp.float32), axis=0) / n_tok
        aux_ref[...] = jnp.reshape(
            jnp.sum(importance * load) * NUM_EXPERTS, (1, 1))
        y_ref[...] = jnp.zeros_like(y_ref)

    w1 = w1_ref[0]
    w2 = w2_ref[0]
    b1 = b1_ref[0]
    b2 = b2_ref[0]
    lane = jax.lax.broadcasted_iota(jnp.int32, (CHUNK, NUM_EXPERTS), 1)

    def body(i, _):
        sl = pl.ds(i * CHUNK, CHUNK)
        w_col = jnp.sum(jnp.where(lane == e, comb_ref[sl, :], 0.0),
                        axis=-1, keepdims=True)
        xs = x_ref[sl, :]
        h = jax.lax.dot_general(xs, w1, (((1,), (1,)), ((), ()))) + b1
        h = h * jax.nn.sigmoid(h)
        ys = jax.lax.dot_general(h, w2, (((1,), (1,)), ((), ()))) + b2
        y_ref[sl, :] += ys * w_col
        return 0

    jax.lax.fori_loop(0, n_tok // CHUNK, body, 0)


def kernel(x, router_W, fc1_W, fc1_b, fc2_W, fc2_b):
    B, L, D = x.shape
    N = B * L
    E, H = fc1_W.shape[0], fc1_W.shape[1]
    x_flat = x.reshape(N, D)

    y, aux = pl.pallas_call(
        _moe_dense_kernel,
        grid=(E,),
        in_specs=[
            pl.BlockSpec((N, D), lambda e: (0, 0)),
            pl.BlockSpec((E, D), lambda e: (0, 0)),
            pl.BlockSpec((1, H, D), lambda e: (e, 0, 0)),
            pl.BlockSpec((1, 1, H), lambda e: (e, 0, 0)),
            pl.BlockSpec((1, D, H), lambda e: (e, 0, 0)),
            pl.BlockSpec((1, 1, D), lambda e: (e, 0, 0)),
        ],
        out_specs=[
            pl.BlockSpec((N, D), lambda e: (0, 0)),
            pl.BlockSpec((1, 1), lambda e: (0, 0)),
        ],
        out_shape=[
            jax.ShapeDtypeStruct((N, D), jnp.float32),
            jax.ShapeDtypeStruct((1, 1), jnp.float32),
        ],
        scratch_shapes=[pltpu.VMEM((N, NUM_EXPERTS), jnp.float32)],
    )(x_flat, router_W, fc1_W, fc1_b.reshape(E, 1, H),
      fc2_W, fc2_b.reshape(E, 1, D))

    return y.reshape(B, L, D), aux[0, 0]
